# baseline (speedup 1.0000x reference)
import jax
import jax.numpy as jnp
from jax import lax
from jax.experimental import pallas as pl
from jax.experimental.pallas import tpu as pltpu
from jax.experimental.pallas import tpu_sc as plsc

B = 16384
D = 32
NC = 2
NS = 16
NW = NC * NS
B_PER_W = B // NW
IDX_CHUNK = 128
N_CHUNKS = B_PER_W // IDX_CHUNK


def _sc_gather_body(uids_hbm, iids_hbm, ug_hbm, ig_hbm, um_hbm, im_hbm,
                    ug_out, ig_out, um_out, im_out,
                    uidx_v, iidx_v, ug_v, ig_v, um_v, im_v, sem):
    wid = lax.axis_index("s") * NC + lax.axis_index("c")
    pltpu.sync_copy(uids_hbm.at[wid], uidx_v)
    pltpu.sync_copy(iids_hbm.at[wid], iidx_v)
    copies = []
    for j in range(N_CHUNKS):
        rows = pl.ds(j * IDX_CHUNK, IDX_CHUNK)
        copies.append(pltpu.async_copy(ug_hbm.at[uidx_v.at[j]], ug_v.at[rows], sem))
        copies.append(pltpu.async_copy(ig_hbm.at[iidx_v.at[j]], ig_v.at[rows], sem))
        copies.append(pltpu.async_copy(um_hbm.at[uidx_v.at[j]], um_v.at[rows], sem))
        copies.append(pltpu.async_copy(im_hbm.at[iidx_v.at[j]], im_v.at[rows], sem))
    for c in copies:
        c.wait()
    base = wid * B_PER_W
    out_rows = pl.ds(base, B_PER_W)
    pltpu.sync_copy(ug_v, ug_out.at[out_rows])
    pltpu.sync_copy(ig_v, ig_out.at[out_rows])
    pltpu.sync_copy(um_v, um_out.at[out_rows])
    pltpu.sync_copy(im_v, im_out.at[out_rows])


def _sc_gather(user_ids, item_ids, user_gmf, item_gmf, user_mlp, item_mlp):
    mesh = plsc.VectorSubcoreMesh(core_axis_name="c", subcore_axis_name="s")
    f32 = jnp.float32
    out_type = tuple(jax.ShapeDtypeStruct((B, D), f32) for _ in range(4))
    scratch = [
        pltpu.VMEM((N_CHUNKS, IDX_CHUNK), jnp.int32),
        pltpu.VMEM((N_CHUNKS, IDX_CHUNK), jnp.int32),
        pltpu.VMEM((B_PER_W, D), f32),
        pltpu.VMEM((B_PER_W, D), f32),
        pltpu.VMEM((B_PER_W, D), f32),
        pltpu.VMEM((B_PER_W, D), f32),
        pltpu.SemaphoreType.DMA,
    ]
    uids = user_ids.astype(jnp.int32).reshape(NW, N_CHUNKS, IDX_CHUNK)
    iids = item_ids.astype(jnp.int32).reshape(NW, N_CHUNKS, IDX_CHUNK)
    run = pl.kernel(_sc_gather_body, out_type=out_type, mesh=mesh,
                    scratch_types=scratch,
                    compiler_params=pltpu.CompilerParams(
                        use_tc_tiling_on_sc=False))
    return run(uids, iids, user_gmf, item_gmf, user_mlp, item_mlp)


def _mlp_body(ug_ref, ig_ref, um_ref, im_ref,
              W1u_ref, W1i_ref, b1_ref, W2_ref, b2_ref, W3_ref, b3_ref,
              Wfg_ref, Wfh_ref, bf_ref, out_ref):
    f32 = jnp.float32
    gmf = ug_ref[...] * ig_ref[...]
    h = jnp.dot(um_ref[...], W1u_ref[...], preferred_element_type=f32)
    h += jnp.dot(im_ref[...], W1i_ref[...], preferred_element_type=f32)
    h = jnp.maximum(h + b1_ref[...], 0.0)
    h = jnp.maximum(jnp.dot(h, W2_ref[...], preferred_element_type=f32)
                    + b2_ref[...], 0.0)
    h = jnp.maximum(jnp.dot(h, W3_ref[...], preferred_element_type=f32)
                    + b3_ref[...], 0.0)
    out = jnp.dot(gmf, Wfg_ref[...], preferred_element_type=f32)
    out += jnp.dot(h, Wfh_ref[...], preferred_element_type=f32)
    out_ref[...] = out + bf_ref[...]


def _mlp(ug, ig, um, im, W1, b1, W2, b2, W3, b3, Wf, bf):
    n_blk = 8
    blk = B // n_blk
    full = lambda shape: pl.BlockSpec(shape, lambda i: (0, 0))
    grid_spec = pl.GridSpec(
        grid=(n_blk,),
        in_specs=[
            pl.BlockSpec((blk, D), lambda i: (i, 0)),
            pl.BlockSpec((blk, D), lambda i: (i, 0)),
            pl.BlockSpec((blk, D), lambda i: (i, 0)),
            pl.BlockSpec((blk, D), lambda i: (i, 0)),
            full((D, 64)), full((D, 64)), full((1, 64)),
            full((64, 32)), full((1, 32)),
            full((32, 16)), full((1, 16)),
            full((D, 1)), full((16, 1)), full((1, 1)),
        ],
        out_specs=pl.BlockSpec((blk, 1), lambda i: (i, 0)),
    )
    return pl.pallas_call(
        _mlp_body,
        grid_spec=grid_spec,
        out_shape=jax.ShapeDtypeStruct((B, 1), jnp.float32),
    )(ug, ig, um, im,
      W1[:D], W1[D:], b1.reshape(1, -1),
      W2, b2.reshape(1, -1),
      W3, b3.reshape(1, -1),
      Wf[:D], Wf[D:], bf.reshape(1, 1))


def kernel(user_ids, item_ids, user_gmf, item_gmf, user_mlp, item_mlp,
           W1, b1, W2, b2, W3, b3, Wf, bf):
    ug, ig, um, im = _sc_gather(user_ids, item_ids,
                                user_gmf, item_gmf, user_mlp, item_mlp)
    return _mlp(ug, ig, um, im, W1, b1, W2, b2, W3, b3, Wf, bf)


# speedup vs baseline: 1.5733x; 1.5733x over previous
import jax
import jax.numpy as jnp
from jax import lax
from jax.experimental import pallas as pl
from jax.experimental.pallas import tpu as pltpu
from jax.experimental.pallas import tpu_sc as plsc

B = 16384
D = 32
GRP = 128 // D
V = 1000000
NC = 2
NS = 16
NW = NC * NS
B_PER_W = B // NW
IDX_CHUNK = 128
N_CHUNKS = B_PER_W // IDX_CHUNK
BLK_I = 8192
SUB = BLK_I // GRP
N_BLK = (V + BLK_I - 1) // BLK_I
VP = N_BLK * SUB


def _repack_body(xt_ref, out_ref):
    x = xt_ref[...]
    for c in range(GRP):
        piece = x[:, c * SUB:(c + 1) * SUB]
        out_ref[:, c * D:(c + 1) * D] = jnp.transpose(piece)


def _repack(table_t):
    return pl.pallas_call(
        _repack_body,
        grid=(N_BLK,),
        in_specs=[pl.BlockSpec((D, BLK_I), lambda i: (0, i))],
        out_specs=pl.BlockSpec((SUB, GRP * D), lambda i: (i, 0)),
        out_shape=jax.ShapeDtypeStruct((VP, GRP * D), jnp.float32),
    )(table_t)


def _sc_gather_body(qids_hbm, tbl_hbm, out_hbm, idx_v, rows_v, sem):
    wid = lax.axis_index("s") * NC + lax.axis_index("c")
    pltpu.sync_copy(qids_hbm.at[wid], idx_v)
    copies = []
    for j in range(N_CHUNKS):
        rows = pl.ds(j * IDX_CHUNK, IDX_CHUNK)
        copies.append(pltpu.async_copy(tbl_hbm.at[idx_v.at[j]],
                                       rows_v.at[rows], sem))
    for c in copies:
        c.wait()
    pltpu.sync_copy(rows_v, out_hbm.at[pl.ds(wid * B_PER_W, B_PER_W)])


def _sc_gather(qids, tbl):
    mesh = plsc.VectorSubcoreMesh(core_axis_name="c", subcore_axis_name="s")
    run = pl.kernel(
        _sc_gather_body,
        out_type=jax.ShapeDtypeStruct((B, GRP * D), jnp.float32),
        mesh=mesh,
        scratch_types=[
            pltpu.VMEM((N_CHUNKS, IDX_CHUNK), jnp.int32),
            pltpu.VMEM((B_PER_W, GRP * D), jnp.float32),
            pltpu.SemaphoreType.DMA,
        ],
    )
    return run(qids, tbl)


def _mlp_body(usel_ref, isel_ref, ug_ref, ig_ref, um_ref, im_ref,
              W1u_ref, W1i_ref, b1_ref, W2_ref, b2_ref, W3_ref, b3_ref,
              Wfg_ref, Wfh_ref, bf_ref, out_ref):
    f32 = jnp.float32
    usel = usel_ref[...]
    isel = isel_ref[...]
    ug128, ig128 = ug_ref[...], ig_ref[...]
    um128, im128 = um_ref[...], im_ref[...]
    ug, um = ug128[:, :D], um128[:, :D]
    ig, im = ig128[:, :D], im128[:, :D]
    for c in range(1, GRP):
        w = slice(c * D, (c + 1) * D)
        umask = usel == c
        imask = isel == c
        ug = jnp.where(umask, ug128[:, w], ug)
        um = jnp.where(umask, um128[:, w], um)
        ig = jnp.where(imask, ig128[:, w], ig)
        im = jnp.where(imask, im128[:, w], im)
    gmf = ug * ig
    h = jnp.dot(um, W1u_ref[...], preferred_element_type=f32)
    h += jnp.dot(im, W1i_ref[...], preferred_element_type=f32)
    h = jnp.maximum(h + b1_ref[...], 0.0)
    h = jnp.maximum(jnp.dot(h, W2_ref[...], preferred_element_type=f32)
                    + b2_ref[...], 0.0)
    h = jnp.maximum(jnp.dot(h, W3_ref[...], preferred_element_type=f32)
                    + b3_ref[...], 0.0)
    out = jnp.dot(gmf, Wfg_ref[...], preferred_element_type=f32)
    out += jnp.dot(h, Wfh_ref[...], preferred_element_type=f32)
    out_ref[...] = out + bf_ref[...]


def _mlp(usel, isel, ug, ig, um, im, W1, b1, W2, b2, W3, b3, Wf, bf):
    n_blk = 8
    blk = B // n_blk
    full = lambda shape: pl.BlockSpec(shape, lambda i: (0, 0))
    row = lambda w: pl.BlockSpec((blk, w), lambda i: (i, 0))
    grid_spec = pl.GridSpec(
        grid=(n_blk,),
        in_specs=[
            row(1), row(1),
            row(128), row(128), row(128), row(128),
            full((D, 64)), full((D, 64)), full((1, 64)),
            full((64, 32)), full((1, 32)),
            full((32, 16)), full((1, 16)),
            full((D, 1)), full((16, 1)), full((1, 1)),
        ],
        out_specs=pl.BlockSpec((blk, 1), lambda i: (i, 0)),
    )
    return pl.pallas_call(
        _mlp_body,
        grid_spec=grid_spec,
        out_shape=jax.ShapeDtypeStruct((B, 1), jnp.float32),
    )(usel, isel, ug, ig, um, im,
      W1[:D], W1[D:], b1.reshape(1, -1),
      W2, b2.reshape(1, -1),
      W3, b3.reshape(1, -1),
      Wf[:D], Wf[D:], bf.reshape(1, 1))


def kernel(user_ids, item_ids, user_gmf, item_gmf, user_mlp, item_mlp,
           W1, b1, W2, b2, W3, b3, Wf, bf):
    uids = user_ids.astype(jnp.int32)
    iids = item_ids.astype(jnp.int32)
    uq = ((uids // BLK_I) * SUB + uids % SUB).reshape(NW, N_CHUNKS, IDX_CHUNK)
    iq = ((iids // BLK_I) * SUB + iids % SUB).reshape(NW, N_CHUNKS, IDX_CHUNK)
    usel = ((uids // SUB) % GRP).reshape(B, 1)
    isel = ((iids // SUB) % GRP).reshape(B, 1)
    gathered = [_sc_gather(q, _repack(t.T))
                for q, t in ((uq, user_gmf), (iq, item_gmf),
                             (uq, user_mlp), (iq, item_mlp))]
    ug, ig, um, im = gathered
    return _mlp(usel, isel, ug, ig, um, im, W1, b1, W2, b2, W3, b3, Wf, bf)


# speedup vs baseline: 2.6451x; 1.6813x over previous
import jax
import jax.numpy as jnp
from jax import lax
from jax.experimental import pallas as pl
from jax.experimental.pallas import tpu as pltpu
from jax.experimental.pallas import tpu_sc as plsc

B = 16384
D = 32
GRP = 128 // D
V = 1000000
NC = 2
NS = 16
NW = NC * NS
B_PER_W = B // NW
IDX_CHUNK = 128
N_CHUNKS = B_PER_W // IDX_CHUNK
BLK_I = 8192
SUB = BLK_I // GRP
N_BLK = (V + BLK_I - 1) // BLK_I
VP = N_BLK * SUB


def _repack_body(xt_ref, out_ref):
    x = xt_ref[...]
    for k in range(SUB // 128):
        xv = jnp.concatenate(
            [x[:, c * SUB + k * 128:c * SUB + k * 128 + 128]
             for c in range(GRP)], axis=0)
        out_ref[k * 128:(k + 1) * 128, :] = jnp.transpose(xv)


def _repack(table_t):
    return pl.pallas_call(
        _repack_body,
        grid=(N_BLK,),
        in_specs=[pl.BlockSpec((D, BLK_I), lambda i: (0, i))],
        out_specs=pl.BlockSpec((SUB, GRP * D), lambda i: (i, 0)),
        out_shape=jax.ShapeDtypeStruct((VP, GRP * D), jnp.float32),
    )(table_t)


def _sc_gather_body(qids_hbm, tbl_hbm, out_hbm, idx_v, rows_v, sem):
    wid = lax.axis_index("s") * NC + lax.axis_index("c")
    pltpu.sync_copy(qids_hbm.at[wid], idx_v)
    copies = []
    for j in range(N_CHUNKS):
        rows = pl.ds(j * IDX_CHUNK, IDX_CHUNK)
        copies.append(pltpu.async_copy(tbl_hbm.at[idx_v.at[j]],
                                       rows_v.at[rows], sem))
    for c in copies:
        c.wait()
    pltpu.sync_copy(rows_v, out_hbm.at[pl.ds(wid * B_PER_W, B_PER_W)])


def _sc_gather(qids, tbl):
    mesh = plsc.VectorSubcoreMesh(core_axis_name="c", subcore_axis_name="s")
    run = pl.kernel(
        _sc_gather_body,
        out_type=jax.ShapeDtypeStruct((B, GRP * D), jnp.float32),
        mesh=mesh,
        scratch_types=[
            pltpu.VMEM((N_CHUNKS, IDX_CHUNK), jnp.int32),
            pltpu.VMEM((B_PER_W, GRP * D), jnp.float32),
            pltpu.SemaphoreType.DMA,
        ],
    )
    return run(qids, tbl)


def _mlp_body(usel_ref, isel_ref, ug_ref, ig_ref, um_ref, im_ref,
              W1u_ref, W1i_ref, b1_ref, W2_ref, b2_ref, W3_ref, b3_ref,
              Wfg_ref, Wfh_ref, bf_ref, out_ref):
    f32 = jnp.float32
    usel = usel_ref[...]
    isel = isel_ref[...]
    ug128, ig128 = ug_ref[...], ig_ref[...]
    um128, im128 = um_ref[...], im_ref[...]
    ug, um = ug128[:, :D], um128[:, :D]
    ig, im = ig128[:, :D], im128[:, :D]
    for c in range(1, GRP):
        w = slice(c * D, (c + 1) * D)
        umask = usel == c
        imask = isel == c
        ug = jnp.where(umask, ug128[:, w], ug)
        um = jnp.where(umask, um128[:, w], um)
        ig = jnp.where(imask, ig128[:, w], ig)
        im = jnp.where(imask, im128[:, w], im)
    gmf = ug * ig
    h = jnp.dot(um, W1u_ref[...], preferred_element_type=f32)
    h += jnp.dot(im, W1i_ref[...], preferred_element_type=f32)
    h = jnp.maximum(h + b1_ref[...], 0.0)
    h = jnp.maximum(jnp.dot(h, W2_ref[...], preferred_element_type=f32)
                    + b2_ref[...], 0.0)
    h = jnp.maximum(jnp.dot(h, W3_ref[...], preferred_element_type=f32)
                    + b3_ref[...], 0.0)
    out = jnp.dot(gmf, Wfg_ref[...], preferred_element_type=f32)
    out += jnp.dot(h, Wfh_ref[...], preferred_element_type=f32)
    out_ref[...] = out + bf_ref[...]


def _mlp(usel, isel, ug, ig, um, im, W1, b1, W2, b2, W3, b3, Wf, bf):
    n_blk = 8
    blk = B // n_blk
    full = lambda shape: pl.BlockSpec(shape, lambda i: (0, 0))
    row = lambda w: pl.BlockSpec((blk, w), lambda i: (i, 0))
    grid_spec = pl.GridSpec(
        grid=(n_blk,),
        in_specs=[
            row(1), row(1),
            row(128), row(128), row(128), row(128),
            full((D, 64)), full((D, 64)), full((1, 64)),
            full((64, 32)), full((1, 32)),
            full((32, 16)), full((1, 16)),
            full((D, 1)), full((16, 1)), full((1, 1)),
        ],
        out_specs=pl.BlockSpec((blk, 1), lambda i: (i, 0)),
    )
    return pl.pallas_call(
        _mlp_body,
        grid_spec=grid_spec,
        out_shape=jax.ShapeDtypeStruct((B, 1), jnp.float32),
    )(usel, isel, ug, ig, um, im,
      W1[:D], W1[D:], b1.reshape(1, -1),
      W2, b2.reshape(1, -1),
      W3, b3.reshape(1, -1),
      Wf[:D], Wf[D:], bf.reshape(1, 1))


def kernel(user_ids, item_ids, user_gmf, item_gmf, user_mlp, item_mlp,
           W1, b1, W2, b2, W3, b3, Wf, bf):
    uids = user_ids.astype(jnp.int32)
    iids = item_ids.astype(jnp.int32)
    uq = ((uids // BLK_I) * SUB + uids % SUB).reshape(NW, N_CHUNKS, IDX_CHUNK)
    iq = ((iids // BLK_I) * SUB + iids % SUB).reshape(NW, N_CHUNKS, IDX_CHUNK)
    usel = ((uids // SUB) % GRP).reshape(B, 1)
    isel = ((iids // SUB) % GRP).reshape(B, 1)
    gathered = [_sc_gather(q, _repack(t.T))
                for q, t in ((uq, user_gmf), (iq, item_gmf),
                             (uq, user_mlp), (iq, item_mlp))]
    ug, ig, um, im = gathered
    return _mlp(usel, isel, ug, ig, um, im, W1, b1, W2, b2, W3, b3, Wf, bf)


# speedup vs baseline: 3.0548x; 1.1549x over previous
import jax
import jax.numpy as jnp
from jax import lax
from jax.experimental import pallas as pl
from jax.experimental.pallas import tpu as pltpu
from jax.experimental.pallas import tpu_sc as plsc

B = 16384
D = 32
HALF = D // 2
GRP = 8
V = 1000000
NC = 2
NS = 16
NW = NC * NS
B_PER_W = B // NW
IDX_CHUNK = 128
N_CHUNKS = B_PER_W // IDX_CHUNK
BLK_I = 8192
SUB = BLK_I // GRP
N_BLK = (V + BLK_I - 1) // BLK_I
VP = N_BLK * SUB


def _repack_body(xt_ref, out_ref):
    x = xt_ref[...]
    xi = lax.bitcast_convert_type(x, jnp.uint32)
    lsb = (xi >> 16) & jnp.uint32(1)
    xr = (xi + jnp.uint32(0x7FFF) + lsb) >> 16
    packed = (xr[:HALF] << 16) | xr[HALF:]
    pf = lax.bitcast_convert_type(packed, jnp.float32)
    for k in range(SUB // 128):
        xv = jnp.concatenate(
            [pf[:, c * SUB + k * 128:c * SUB + k * 128 + 128]
             for c in range(GRP)], axis=0)
        out_ref[k * 128:(k + 1) * 128, :] = jnp.transpose(xv)


def _repack(table_t):
    return pl.pallas_call(
        _repack_body,
        grid=(N_BLK,),
        in_specs=[pl.BlockSpec((D, BLK_I), lambda i: (0, i))],
        out_specs=pl.BlockSpec((SUB, GRP * HALF), lambda i: (i, 0)),
        out_shape=jax.ShapeDtypeStruct((VP, GRP * HALF), jnp.float32),
    )(table_t)


def _sc_gather_body(qids_hbm, tbl_hbm, out_hbm, idx_v, rows_v, sem):
    wid = lax.axis_index("s") * NC + lax.axis_index("c")
    pltpu.sync_copy(qids_hbm.at[wid], idx_v)
    copies = []
    for j in range(N_CHUNKS):
        rows = pl.ds(j * IDX_CHUNK, IDX_CHUNK)
        copies.append(pltpu.async_copy(tbl_hbm.at[idx_v.at[j]],
                                       rows_v.at[rows], sem))
    for c in copies:
        c.wait()
    pltpu.sync_copy(rows_v, out_hbm.at[pl.ds(wid * B_PER_W, B_PER_W)])


def _sc_gather(qids, tbl):
    mesh = plsc.VectorSubcoreMesh(core_axis_name="c", subcore_axis_name="s")
    run = pl.kernel(
        _sc_gather_body,
        out_type=jax.ShapeDtypeStruct((B, GRP * HALF), jnp.float32),
        mesh=mesh,
        scratch_types=[
            pltpu.VMEM((N_CHUNKS, IDX_CHUNK), jnp.int32),
            pltpu.VMEM((B_PER_W, GRP * HALF), jnp.float32),
            pltpu.SemaphoreType.DMA,
        ],
    )
    return run(qids, tbl)


def _unpack_sel(x128, sel):
    xi = lax.bitcast_convert_type(x128, jnp.uint32)
    lane_w = lax.broadcasted_iota(jnp.int32, xi.shape, 1) // HALF
    z = jnp.where(lane_w == sel, xi, jnp.uint32(0))
    z = z[:, :64] | z[:, 64:]
    z = z[:, :32] | z[:, 32:]
    w = z[:, :HALF] | z[:, HALF:]
    hi = lax.bitcast_convert_type(w & jnp.uint32(0xFFFF0000), jnp.float32)
    lo = lax.bitcast_convert_type(w << 16, jnp.float32)
    return jnp.concatenate([hi, lo], axis=1)


def _mlp_body(usel_ref, isel_ref, ug_ref, ig_ref, um_ref, im_ref,
              W1u_ref, W1i_ref, b1_ref, W2_ref, b2_ref, W3_ref, b3_ref,
              Wfg_ref, Wfh_ref, bf_ref, out_ref):
    f32 = jnp.float32
    usel = usel_ref[...]
    isel = isel_ref[...]
    ug = _unpack_sel(ug_ref[...], usel)
    um = _unpack_sel(um_ref[...], usel)
    ig = _unpack_sel(ig_ref[...], isel)
    im = _unpack_sel(im_ref[...], isel)
    gmf = ug * ig
    h = jnp.dot(um, W1u_ref[...], preferred_element_type=f32)
    h += jnp.dot(im, W1i_ref[...], preferred_element_type=f32)
    h = jnp.maximum(h + b1_ref[...], 0.0)
    h = jnp.maximum(jnp.dot(h, W2_ref[...], preferred_element_type=f32)
                    + b2_ref[...], 0.0)
    h = jnp.maximum(jnp.dot(h, W3_ref[...], preferred_element_type=f32)
                    + b3_ref[...], 0.0)
    out = jnp.dot(gmf, Wfg_ref[...], preferred_element_type=f32)
    out += jnp.dot(h, Wfh_ref[...], preferred_element_type=f32)
    out_ref[...] = out + bf_ref[...]


def _mlp(usel, isel, ug, ig, um, im, W1, b1, W2, b2, W3, b3, Wf, bf):
    n_blk = 8
    blk = B // n_blk
    full = lambda shape: pl.BlockSpec(shape, lambda i: (0, 0))
    row = lambda w: pl.BlockSpec((blk, w), lambda i: (i, 0))
    grid_spec = pl.GridSpec(
        grid=(n_blk,),
        in_specs=[
            row(1), row(1),
            row(128), row(128), row(128), row(128),
            full((D, 64)), full((D, 64)), full((1, 64)),
            full((64, 32)), full((1, 32)),
            full((32, 16)), full((1, 16)),
            full((D, 1)), full((16, 1)), full((1, 1)),
        ],
        out_specs=pl.BlockSpec((blk, 1), lambda i: (i, 0)),
    )
    return pl.pallas_call(
        _mlp_body,
        grid_spec=grid_spec,
        out_shape=jax.ShapeDtypeStruct((B, 1), jnp.float32),
    )(usel, isel, ug, ig, um, im,
      W1[:D], W1[D:], b1.reshape(1, -1),
      W2, b2.reshape(1, -1),
      W3, b3.reshape(1, -1),
      Wf[:D], Wf[D:], bf.reshape(1, 1))


def kernel(user_ids, item_ids, user_gmf, item_gmf, user_mlp, item_mlp,
           W1, b1, W2, b2, W3, b3, Wf, bf):
    uids = user_ids.astype(jnp.int32)
    iids = item_ids.astype(jnp.int32)
    uq = ((uids // BLK_I) * SUB + uids % SUB).reshape(NW, N_CHUNKS, IDX_CHUNK)
    iq = ((iids // BLK_I) * SUB + iids % SUB).reshape(NW, N_CHUNKS, IDX_CHUNK)
    usel = ((uids // SUB) % GRP).reshape(B, 1)
    isel = ((iids // SUB) % GRP).reshape(B, 1)
    gathered = [_sc_gather(q, _repack(t.T))
                for q, t in ((uq, user_gmf), (iq, item_gmf),
                             (uq, user_mlp), (iq, item_mlp))]
    ug, ig, um, im = gathered
    return _mlp(usel, isel, ug, ig, um, im, W1, b1, W2, b2, W3, b3, Wf, bf)


# speedup vs baseline: 3.9217x; 1.2838x over previous
import jax
import jax.numpy as jnp
from jax import lax
from jax.experimental import pallas as pl
from jax.experimental.pallas import tpu as pltpu
from jax.experimental.pallas import tpu_sc as plsc

B = 16384
D = 32
HALF = D // 2
GRP = 8
V = 1000000
NC = 2
NS = 16
NW = NC * NS
B_PER_W = B // NW
IDX_CHUNK = 128
N_CHUNKS = B_PER_W // IDX_CHUNK
BLK_I = 16384
SUB = BLK_I // GRP
N_BLK = (V + BLK_I - 1) // BLK_I
VP = N_BLK * SUB


def _repack_body(xt_ref, out_ref):
    x = xt_ref[...]
    xi = lax.bitcast_convert_type(x, jnp.uint32)
    lsb = (xi >> 16) & jnp.uint32(1)
    xr = (xi + jnp.uint32(0x7FFF) + lsb) >> 16
    packed = (xr[:HALF] << 16) | xr[HALF:]
    pf = lax.bitcast_convert_type(packed, jnp.float32)
    for k in range(SUB // 128):
        xv = jnp.concatenate(
            [pf[:, c * SUB + k * 128:c * SUB + k * 128 + 128]
             for c in range(GRP)], axis=0)
        out_ref[k * 128:(k + 1) * 128, :] = jnp.transpose(xv)


def _repack(table_t):
    return pl.pallas_call(
        _repack_body,
        grid=(N_BLK,),
        in_specs=[pl.BlockSpec((D, BLK_I), lambda i: (0, i))],
        out_specs=pl.BlockSpec((SUB, GRP * HALF), lambda i: (i, 0)),
        out_shape=jax.ShapeDtypeStruct((VP, GRP * HALF), jnp.float32),
    )(table_t)


def _sc_gather_body(qids_hbm, tbl_hbm, out_hbm, idx_v, rows_v, sem):
    wid = lax.axis_index("s") * NC + lax.axis_index("c")
    pltpu.sync_copy(qids_hbm.at[wid], idx_v)
    copies = []
    for j in range(N_CHUNKS):
        rows = pl.ds(j * IDX_CHUNK, IDX_CHUNK)
        copies.append(pltpu.async_copy(tbl_hbm.at[idx_v.at[j]],
                                       rows_v.at[rows], sem))
    for c in copies:
        c.wait()
    pltpu.sync_copy(rows_v, out_hbm.at[pl.ds(wid * B_PER_W, B_PER_W)])


def _sc_gather(qids, tbl):
    mesh = plsc.VectorSubcoreMesh(core_axis_name="c", subcore_axis_name="s")
    run = pl.kernel(
        _sc_gather_body,
        out_type=jax.ShapeDtypeStruct((B, GRP * HALF), jnp.float32),
        mesh=mesh,
        scratch_types=[
            pltpu.VMEM((N_CHUNKS, IDX_CHUNK), jnp.int32),
            pltpu.VMEM((B_PER_W, GRP * HALF), jnp.float32),
            pltpu.SemaphoreType.DMA,
        ],
    )
    return run(qids, tbl)


def _unpack_sel(x128, sel):
    xi = lax.bitcast_convert_type(x128, jnp.uint32)
    lane_w = lax.broadcasted_iota(jnp.int32, xi.shape, 1) // HALF
    z = jnp.where(lane_w == sel, xi, jnp.uint32(0))
    z = z[:, :64] | z[:, 64:]
    z = z[:, :32] | z[:, 32:]
    w = z[:, :HALF] | z[:, HALF:]
    hi = lax.bitcast_convert_type(w & jnp.uint32(0xFFFF0000), jnp.float32)
    lo = lax.bitcast_convert_type(w << 16, jnp.float32)
    return jnp.concatenate([hi, lo], axis=1)


def _mlp_body(usel_ref, isel_ref, ug_ref, ig_ref, um_ref, im_ref,
              W1u_ref, W1i_ref, b1_ref, W2_ref, b2_ref, W3_ref, b3_ref,
              Wfg_ref, Wfh_ref, bf_ref, out_ref):
    f32 = jnp.float32
    usel = usel_ref[...]
    isel = isel_ref[...]
    ug = _unpack_sel(ug_ref[...], usel)
    um = _unpack_sel(um_ref[...], usel)
    ig = _unpack_sel(ig_ref[...], isel)
    im = _unpack_sel(im_ref[...], isel)
    gmf = ug * ig
    h = jnp.dot(um, W1u_ref[...], preferred_element_type=f32)
    h += jnp.dot(im, W1i_ref[...], preferred_element_type=f32)
    h = jnp.maximum(h + b1_ref[...], 0.0)
    h = jnp.maximum(jnp.dot(h, W2_ref[...], preferred_element_type=f32)
                    + b2_ref[...], 0.0)
    h = jnp.maximum(jnp.dot(h, W3_ref[...], preferred_element_type=f32)
                    + b3_ref[...], 0.0)
    out = jnp.dot(gmf, Wfg_ref[...], preferred_element_type=f32)
    out += jnp.dot(h, Wfh_ref[...], preferred_element_type=f32)
    out_ref[...] = out + bf_ref[...]


def _mlp(usel, isel, ug, ig, um, im, W1, b1, W2, b2, W3, b3, Wf, bf):
    n_blk = 4
    blk = B // n_blk
    full = lambda shape: pl.BlockSpec(shape, lambda i: (0, 0))
    row = lambda w: pl.BlockSpec((blk, w), lambda i: (i, 0))
    grid_spec = pl.GridSpec(
        grid=(n_blk,),
        in_specs=[
            row(1), row(1),
            row(128), row(128), row(128), row(128),
            full((D, 64)), full((D, 64)), full((1, 64)),
            full((64, 32)), full((1, 32)),
            full((32, 16)), full((1, 16)),
            full((D, 1)), full((16, 1)), full((1, 1)),
        ],
        out_specs=pl.BlockSpec((blk, 1), lambda i: (i, 0)),
    )
    return pl.pallas_call(
        _mlp_body,
        grid_spec=grid_spec,
        out_shape=jax.ShapeDtypeStruct((B, 1), jnp.float32),
    )(usel, isel, ug, ig, um, im,
      W1[:D], W1[D:], b1.reshape(1, -1),
      W2, b2.reshape(1, -1),
      W3, b3.reshape(1, -1),
      Wf[:D], Wf[D:], bf.reshape(1, 1))


def kernel(user_ids, item_ids, user_gmf, item_gmf, user_mlp, item_mlp,
           W1, b1, W2, b2, W3, b3, Wf, bf):
    uids = user_ids.astype(jnp.int32)
    iids = item_ids.astype(jnp.int32)
    uq = ((uids // BLK_I) * SUB + uids % SUB).reshape(NW, N_CHUNKS, IDX_CHUNK)
    iq = ((iids // BLK_I) * SUB + iids % SUB).reshape(NW, N_CHUNKS, IDX_CHUNK)
    usel = ((uids // SUB) % GRP).reshape(B, 1)
    isel = ((iids // SUB) % GRP).reshape(B, 1)
    gathered = [_sc_gather(q, _repack(t.T))
                for q, t in ((uq, user_gmf), (iq, item_gmf),
                             (uq, user_mlp), (iq, item_mlp))]
    ug, ig, um, im = gathered
    return _mlp(usel, isel, ug, ig, um, im, W1, b1, W2, b2, W3, b3, Wf, bf)


# speedup vs baseline: 4.6413x; 1.1835x over previous
import jax
import jax.numpy as jnp
from jax import lax
from jax.experimental import pallas as pl
from jax.experimental.pallas import tpu as pltpu
from jax.experimental.pallas import tpu_sc as plsc

B = 16384
D = 32
HALF = D // 2
GRP = 8
V = 1000000
NC = 2
NS = 16
NW = NC * NS
B_PER_W = B // NW
IDX_CHUNK = 128
N_CHUNKS = B_PER_W // IDX_CHUNK
BLK_I = 32768
SUB = BLK_I // GRP
N_BLK = (V + BLK_I - 1) // BLK_I
VP = N_BLK * SUB


def _repack_body(xt_ref, out_ref):
    x = xt_ref[...]
    xi = lax.bitcast_convert_type(x, jnp.uint32)
    lsb = (xi >> 16) & jnp.uint32(1)
    xr = (xi + jnp.uint32(0x7FFF) + lsb) >> 16
    packed = (xr[:HALF] << 16) | xr[HALF:]
    pf = lax.bitcast_convert_type(packed, jnp.float32)
    for k in range(SUB // 128):
        xv = jnp.concatenate(
            [pf[:, c * SUB + k * 128:c * SUB + k * 128 + 128]
             for c in range(GRP)], axis=0)
        out_ref[k * 128:(k + 1) * 128, :] = jnp.transpose(xv)


def _repack(table_t):
    return pl.pallas_call(
        _repack_body,
        grid=(N_BLK,),
        in_specs=[pl.BlockSpec((D, BLK_I), lambda i: (0, i))],
        out_specs=pl.BlockSpec((SUB, GRP * HALF), lambda i: (i, 0)),
        out_shape=jax.ShapeDtypeStruct((VP, GRP * HALF), jnp.float32),
    )(table_t)


def _sc_gather_body(qids_hbm, tbl_hbm, out_hbm, idx_v, rows_v, sem):
    wid = lax.axis_index("s") * NC + lax.axis_index("c")
    pltpu.sync_copy(qids_hbm.at[wid], idx_v)
    copies = []
    for j in range(N_CHUNKS):
        rows = pl.ds(j * IDX_CHUNK, IDX_CHUNK)
        copies.append(pltpu.async_copy(tbl_hbm.at[idx_v.at[j]],
                                       rows_v.at[rows], sem))
    for c in copies:
        c.wait()
    pltpu.sync_copy(rows_v, out_hbm.at[pl.ds(wid * B_PER_W, B_PER_W)])


def _sc_gather(qids, tbl):
    mesh = plsc.VectorSubcoreMesh(core_axis_name="c", subcore_axis_name="s")
    run = pl.kernel(
        _sc_gather_body,
        out_type=jax.ShapeDtypeStruct((B, GRP * HALF), jnp.float32),
        mesh=mesh,
        scratch_types=[
            pltpu.VMEM((N_CHUNKS, IDX_CHUNK), jnp.int32),
            pltpu.VMEM((B_PER_W, GRP * HALF), jnp.float32),
            pltpu.SemaphoreType.DMA,
        ],
    )
    return run(qids, tbl)


def _unpack_sel(x128, sel):
    xi = lax.bitcast_convert_type(x128, jnp.uint32)
    lane_w = lax.broadcasted_iota(jnp.int32, xi.shape, 1) // HALF
    z = jnp.where(lane_w == sel, xi, jnp.uint32(0))
    z = z[:, :64] | z[:, 64:]
    z = z[:, :32] | z[:, 32:]
    w = z[:, :HALF] | z[:, HALF:]
    hi = lax.bitcast_convert_type(w & jnp.uint32(0xFFFF0000), jnp.float32)
    lo = lax.bitcast_convert_type(w << 16, jnp.float32)
    return jnp.concatenate([hi, lo], axis=1)


def _mlp_body(usel_ref, isel_ref, ug_ref, ig_ref, um_ref, im_ref,
              W1u_ref, W1i_ref, b1_ref, W2_ref, b2_ref, W3_ref, b3_ref,
              Wfg_ref, Wfh_ref, bf_ref, out_ref):
    f32 = jnp.float32
    usel = usel_ref[...]
    isel = isel_ref[...]
    ug = _unpack_sel(ug_ref[...], usel)
    um = _unpack_sel(um_ref[...], usel)
    ig = _unpack_sel(ig_ref[...], isel)
    im = _unpack_sel(im_ref[...], isel)
    gmf = ug * ig
    h = jnp.dot(um, W1u_ref[...], preferred_element_type=f32)
    h += jnp.dot(im, W1i_ref[...], preferred_element_type=f32)
    h = jnp.maximum(h + b1_ref[...], 0.0)
    h = jnp.maximum(jnp.dot(h, W2_ref[...], preferred_element_type=f32)
                    + b2_ref[...], 0.0)
    h = jnp.maximum(jnp.dot(h, W3_ref[...], preferred_element_type=f32)
                    + b3_ref[...], 0.0)
    out = jnp.dot(gmf, Wfg_ref[...], preferred_element_type=f32)
    out += jnp.dot(h, Wfh_ref[...], preferred_element_type=f32)
    out_ref[...] = out + bf_ref[...]


def _mlp(usel, isel, ug, ig, um, im, W1, b1, W2, b2, W3, b3, Wf, bf):
    n_blk = 4
    blk = B // n_blk
    full = lambda shape: pl.BlockSpec(shape, lambda i: (0, 0))
    row = lambda w: pl.BlockSpec((blk, w), lambda i: (i, 0))
    grid_spec = pl.GridSpec(
        grid=(n_blk,),
        in_specs=[
            row(1), row(1),
            row(128), row(128), row(128), row(128),
            full((D, 64)), full((D, 64)), full((1, 64)),
            full((64, 32)), full((1, 32)),
            full((32, 16)), full((1, 16)),
            full((D, 1)), full((16, 1)), full((1, 1)),
        ],
        out_specs=pl.BlockSpec((blk, 1), lambda i: (i, 0)),
    )
    return pl.pallas_call(
        _mlp_body,
        grid_spec=grid_spec,
        out_shape=jax.ShapeDtypeStruct((B, 1), jnp.float32),
    )(usel, isel, ug, ig, um, im,
      W1[:D], W1[D:], b1.reshape(1, -1),
      W2, b2.reshape(1, -1),
      W3, b3.reshape(1, -1),
      Wf[:D], Wf[D:], bf.reshape(1, 1))


def kernel(user_ids, item_ids, user_gmf, item_gmf, user_mlp, item_mlp,
           W1, b1, W2, b2, W3, b3, Wf, bf):
    uids = user_ids.astype(jnp.int32)
    iids = item_ids.astype(jnp.int32)
    uq = ((uids // BLK_I) * SUB + uids % SUB).reshape(NW, N_CHUNKS, IDX_CHUNK)
    iq = ((iids // BLK_I) * SUB + iids % SUB).reshape(NW, N_CHUNKS, IDX_CHUNK)
    usel = ((uids // SUB) % GRP).reshape(B, 1)
    isel = ((iids // SUB) % GRP).reshape(B, 1)
    gathered = [_sc_gather(q, _repack(t.T))
                for q, t in ((uq, user_gmf), (iq, item_gmf),
                             (uq, user_mlp), (iq, item_mlp))]
    ug, ig, um, im = gathered
    return _mlp(usel, isel, ug, ig, um, im, W1, b1, W2, b2, W3, b3, Wf, bf)


# speedup vs baseline: 4.7485x; 1.0231x over previous
import jax
import jax.numpy as jnp
from jax import lax
from jax.experimental import pallas as pl
from jax.experimental.pallas import tpu as pltpu
from jax.experimental.pallas import tpu_sc as plsc

B = 16384
D = 32
HALF = D // 2
GRP = 8
V = 1000000
NC = 2
NS = 16
NW = NC * NS
B_PER_W = B // NW
IDX_CHUNK = 128
N_CHUNKS = B_PER_W // IDX_CHUNK
BLK_I = 65536
SUB = BLK_I // GRP
N_BLK = (V + BLK_I - 1) // BLK_I
VP = N_BLK * SUB


def _repack_body(xt_ref, out_ref):
    x = xt_ref[...]
    xi = lax.bitcast_convert_type(x, jnp.uint32)
    lsb = (xi >> 16) & jnp.uint32(1)
    xr = (xi + jnp.uint32(0x7FFF) + lsb) >> 16
    packed = (xr[:HALF] << 16) | xr[HALF:]
    pf = lax.bitcast_convert_type(packed, jnp.float32)
    for k in range(SUB // 128):
        xv = jnp.concatenate(
            [pf[:, c * SUB + k * 128:c * SUB + k * 128 + 128]
             for c in range(GRP)], axis=0)
        out_ref[k * 128:(k + 1) * 128, :] = jnp.transpose(xv)


def _repack(table_t):
    return pl.pallas_call(
        _repack_body,
        grid=(N_BLK,),
        in_specs=[pl.BlockSpec((D, BLK_I), lambda i: (0, i))],
        out_specs=pl.BlockSpec((SUB, GRP * HALF), lambda i: (i, 0)),
        out_shape=jax.ShapeDtypeStruct((VP, GRP * HALF), jnp.float32),
    )(table_t)


def _sc_gather_body(qids_hbm, tbl_hbm, out_hbm, idx_v, rows_v, sem):
    wid = lax.axis_index("s") * NC + lax.axis_index("c")
    pltpu.sync_copy(qids_hbm.at[wid], idx_v)
    copies = []
    for j in range(N_CHUNKS):
        rows = pl.ds(j * IDX_CHUNK, IDX_CHUNK)
        copies.append(pltpu.async_copy(tbl_hbm.at[idx_v.at[j]],
                                       rows_v.at[rows], sem))
    for c in copies:
        c.wait()
    pltpu.sync_copy(rows_v, out_hbm.at[pl.ds(wid * B_PER_W, B_PER_W)])


def _sc_gather(qids, tbl):
    mesh = plsc.VectorSubcoreMesh(core_axis_name="c", subcore_axis_name="s")
    run = pl.kernel(
        _sc_gather_body,
        out_type=jax.ShapeDtypeStruct((B, GRP * HALF), jnp.float32),
        mesh=mesh,
        scratch_types=[
            pltpu.VMEM((N_CHUNKS, IDX_CHUNK), jnp.int32),
            pltpu.VMEM((B_PER_W, GRP * HALF), jnp.float32),
            pltpu.SemaphoreType.DMA,
        ],
    )
    return run(qids, tbl)


def _unpack_sel(x128, sel):
    xi = lax.bitcast_convert_type(x128, jnp.uint32)
    lane_w = lax.broadcasted_iota(jnp.int32, xi.shape, 1) // HALF
    z = jnp.where(lane_w == sel, xi, jnp.uint32(0))
    z = z[:, :64] | z[:, 64:]
    z = z[:, :32] | z[:, 32:]
    w = z[:, :HALF] | z[:, HALF:]
    hi = lax.bitcast_convert_type(w & jnp.uint32(0xFFFF0000), jnp.float32)
    lo = lax.bitcast_convert_type(w << 16, jnp.float32)
    return jnp.concatenate([hi, lo], axis=1)


def _mlp_body(usel_ref, isel_ref, ug_ref, ig_ref, um_ref, im_ref,
              W1u_ref, W1i_ref, b1_ref, W2_ref, b2_ref, W3_ref, b3_ref,
              Wfg_ref, Wfh_ref, bf_ref, out_ref):
    f32 = jnp.float32
    usel = usel_ref[...]
    isel = isel_ref[...]
    ug = _unpack_sel(ug_ref[...], usel)
    um = _unpack_sel(um_ref[...], usel)
    ig = _unpack_sel(ig_ref[...], isel)
    im = _unpack_sel(im_ref[...], isel)
    gmf = ug * ig
    h = jnp.dot(um, W1u_ref[...], preferred_element_type=f32)
    h += jnp.dot(im, W1i_ref[...], preferred_element_type=f32)
    h = jnp.maximum(h + b1_ref[...], 0.0)
    h = jnp.maximum(jnp.dot(h, W2_ref[...], preferred_element_type=f32)
                    + b2_ref[...], 0.0)
    h = jnp.maximum(jnp.dot(h, W3_ref[...], preferred_element_type=f32)
                    + b3_ref[...], 0.0)
    out = jnp.dot(gmf, Wfg_ref[...], preferred_element_type=f32)
    out += jnp.dot(h, Wfh_ref[...], preferred_element_type=f32)
    out_ref[...] = out + bf_ref[...]


def _mlp(usel, isel, ug, ig, um, im, W1, b1, W2, b2, W3, b3, Wf, bf):
    n_blk = 4
    blk = B // n_blk
    full = lambda shape: pl.BlockSpec(shape, lambda i: (0, 0))
    row = lambda w: pl.BlockSpec((blk, w), lambda i: (i, 0))
    grid_spec = pl.GridSpec(
        grid=(n_blk,),
        in_specs=[
            row(1), row(1),
            row(128), row(128), row(128), row(128),
            full((D, 64)), full((D, 64)), full((1, 64)),
            full((64, 32)), full((1, 32)),
            full((32, 16)), full((1, 16)),
            full((D, 1)), full((16, 1)), full((1, 1)),
        ],
        out_specs=pl.BlockSpec((blk, 1), lambda i: (i, 0)),
    )
    return pl.pallas_call(
        _mlp_body,
        grid_spec=grid_spec,
        out_shape=jax.ShapeDtypeStruct((B, 1), jnp.float32),
    )(usel, isel, ug, ig, um, im,
      W1[:D], W1[D:], b1.reshape(1, -1),
      W2, b2.reshape(1, -1),
      W3, b3.reshape(1, -1),
      Wf[:D], Wf[D:], bf.reshape(1, 1))


def kernel(user_ids, item_ids, user_gmf, item_gmf, user_mlp, item_mlp,
           W1, b1, W2, b2, W3, b3, Wf, bf):
    uids = user_ids.astype(jnp.int32)
    iids = item_ids.astype(jnp.int32)
    uq = ((uids // BLK_I) * SUB + uids % SUB).reshape(NW, N_CHUNKS, IDX_CHUNK)
    iq = ((iids // BLK_I) * SUB + iids % SUB).reshape(NW, N_CHUNKS, IDX_CHUNK)
    usel = ((uids // SUB) % GRP).reshape(B, 1)
    isel = ((iids // SUB) % GRP).reshape(B, 1)
    gathered = [_sc_gather(q, _repack(t.T))
                for q, t in ((uq, user_gmf), (iq, item_gmf),
                             (uq, user_mlp), (iq, item_mlp))]
    ug, ig, um, im = gathered
    return _mlp(usel, isel, ug, ig, um, im, W1, b1, W2, b2, W3, b3, Wf, bf)


# speedup vs baseline: 5.3190x; 1.1201x over previous
import jax
import jax.numpy as jnp
from jax import lax
from jax.experimental import pallas as pl
from jax.experimental.pallas import tpu as pltpu
from jax.experimental.pallas import tpu_sc as plsc

B = 16384
D = 32
HALF = D // 2
GRP = 4
V = 1000000
NC = 2
NS = 16
NW = NC * NS
B_PER_W = B // NW
IDX_CHUNK = 128
N_CHUNKS = B_PER_W // IDX_CHUNK
BLK_I = 32768
SUB = BLK_I // GRP
N_BLK = (V + BLK_I - 1) // BLK_I
VP = N_BLK * SUB


def _pack16(x):
    xi = lax.bitcast_convert_type(x, jnp.uint32)
    lsb = (xi >> 16) & jnp.uint32(1)
    xr = (xi + jnp.uint32(0x7FFF) + lsb) >> 16
    packed = (xr[:HALF] << 16) | xr[HALF:]
    return lax.bitcast_convert_type(packed, jnp.float32)


def _repack_body(xg_ref, xm_ref, out_ref):
    pg = _pack16(xg_ref[...])
    pm = _pack16(xm_ref[...])
    for k in range(SUB // 128):
        xv = jnp.concatenate(
            [p[:, c * SUB + k * 128:c * SUB + k * 128 + 128]
             for c in range(GRP) for p in (pg, pm)], axis=0)
        out_ref[k * 128:(k + 1) * 128, :] = jnp.transpose(xv)


def _repack(tg_t, tm_t):
    return pl.pallas_call(
        _repack_body,
        grid=(N_BLK,),
        in_specs=[pl.BlockSpec((D, BLK_I), lambda i: (0, i)),
                  pl.BlockSpec((D, BLK_I), lambda i: (0, i))],
        out_specs=pl.BlockSpec((SUB, 128), lambda i: (i, 0)),
        out_shape=jax.ShapeDtypeStruct((VP, 128), jnp.float32),
    )(tg_t, tm_t)


def _sc_gather_body(qids_hbm, tbl_hbm, out_hbm, idx_v, rows_v, sem):
    wid = lax.axis_index("s") * NC + lax.axis_index("c")
    pltpu.sync_copy(qids_hbm.at[wid], idx_v)
    copies = []
    for j in range(N_CHUNKS):
        rows = pl.ds(j * IDX_CHUNK, IDX_CHUNK)
        copies.append(pltpu.async_copy(tbl_hbm.at[idx_v.at[j]],
                                       rows_v.at[rows], sem))
    for c in copies:
        c.wait()
    pltpu.sync_copy(rows_v, out_hbm.at[pl.ds(wid * B_PER_W, B_PER_W)])


def _sc_gather(qids, tbl):
    mesh = plsc.VectorSubcoreMesh(core_axis_name="c", subcore_axis_name="s")
    run = pl.kernel(
        _sc_gather_body,
        out_type=jax.ShapeDtypeStruct((B, 128), jnp.float32),
        mesh=mesh,
        scratch_types=[
            pltpu.VMEM((N_CHUNKS, IDX_CHUNK), jnp.int32),
            pltpu.VMEM((B_PER_W, 128), jnp.float32),
            pltpu.SemaphoreType.DMA,
        ],
    )
    return run(qids, tbl)


def _unpack_sel(x128, sel):
    xi = lax.bitcast_convert_type(x128, jnp.uint32)
    lane_w = lax.broadcasted_iota(jnp.int32, xi.shape, 1) // D
    z = jnp.where(lane_w == sel, xi, jnp.uint32(0))
    z = z[:, :64] | z[:, 64:]
    w = z[:, :D] | z[:, D:]
    hi = lax.bitcast_convert_type(w & jnp.uint32(0xFFFF0000), jnp.float32)
    lo = lax.bitcast_convert_type(w << 16, jnp.float32)
    g = jnp.concatenate([hi[:, :HALF], lo[:, :HALF]], axis=1)
    m = jnp.concatenate([hi[:, HALF:], lo[:, HALF:]], axis=1)
    return g, m


def _mlp_body(usel_ref, isel_ref, gu_ref, gi_ref,
              W1u_ref, W1i_ref, b1_ref, W2_ref, b2_ref, W3_ref, b3_ref,
              Wfg_ref, Wfh_ref, bf_ref, out_ref):
    f32 = jnp.float32
    ug, um = _unpack_sel(gu_ref[...], usel_ref[...])
    ig, im = _unpack_sel(gi_ref[...], isel_ref[...])
    gmf = ug * ig
    h = jnp.dot(um, W1u_ref[...], preferred_element_type=f32)
    h += jnp.dot(im, W1i_ref[...], preferred_element_type=f32)
    h = jnp.maximum(h + b1_ref[...], 0.0)
    h = jnp.maximum(jnp.dot(h, W2_ref[...], preferred_element_type=f32)
                    + b2_ref[...], 0.0)
    h = jnp.maximum(jnp.dot(h, W3_ref[...], preferred_element_type=f32)
                    + b3_ref[...], 0.0)
    out = jnp.dot(gmf, Wfg_ref[...], preferred_element_type=f32)
    out += jnp.dot(h, Wfh_ref[...], preferred_element_type=f32)
    out_ref[...] = out + bf_ref[...]


def _mlp(usel, isel, gu, gi, W1, b1, W2, b2, W3, b3, Wf, bf):
    n_blk = 4
    blk = B // n_blk
    full = lambda shape: pl.BlockSpec(shape, lambda i: (0, 0))
    row = lambda w: pl.BlockSpec((blk, w), lambda i: (i, 0))
    grid_spec = pl.GridSpec(
        grid=(n_blk,),
        in_specs=[
            row(1), row(1),
            row(128), row(128),
            full((D, 64)), full((D, 64)), full((1, 64)),
            full((64, 32)), full((1, 32)),
            full((32, 16)), full((1, 16)),
            full((D, 1)), full((16, 1)), full((1, 1)),
        ],
        out_specs=pl.BlockSpec((blk, 1), lambda i: (i, 0)),
    )
    return pl.pallas_call(
        _mlp_body,
        grid_spec=grid_spec,
        out_shape=jax.ShapeDtypeStruct((B, 1), jnp.float32),
    )(usel, isel, gu, gi,
      W1[:D], W1[D:], b1.reshape(1, -1),
      W2, b2.reshape(1, -1),
      W3, b3.reshape(1, -1),
      Wf[:D], Wf[D:], bf.reshape(1, 1))


def kernel(user_ids, item_ids, user_gmf, item_gmf, user_mlp, item_mlp,
           W1, b1, W2, b2, W3, b3, Wf, bf):
    uids = user_ids.astype(jnp.int32)
    iids = item_ids.astype(jnp.int32)
    uq = ((uids // BLK_I) * SUB + uids % SUB).reshape(NW, N_CHUNKS, IDX_CHUNK)
    iq = ((iids // BLK_I) * SUB + iids % SUB).reshape(NW, N_CHUNKS, IDX_CHUNK)
    usel = ((uids // SUB) % GRP).reshape(B, 1)
    isel = ((iids // SUB) % GRP).reshape(B, 1)
    gu = _sc_gather(uq, _repack(user_gmf.T, user_mlp.T))
    gi = _sc_gather(iq, _repack(item_gmf.T, item_mlp.T))
    return _mlp(usel, isel, gu, gi, W1, b1, W2, b2, W3, b3, Wf, bf)


# speedup vs baseline: 5.3774x; 1.0110x over previous
import jax
import jax.numpy as jnp
from jax import lax
from jax.experimental import pallas as pl
from jax.experimental.pallas import tpu as pltpu
from jax.experimental.pallas import tpu_sc as plsc

B = 16384
D = 32
HALF = D // 2
GRP = 4
V = 1000000
NC = 2
NS = 16
NW = NC * NS
B_PER_W = B // NW
IDX_CHUNK = 128
N_CHUNKS = B_PER_W // IDX_CHUNK
BLK_I = 65536
SUB = BLK_I // GRP
N_BLK = (V + BLK_I - 1) // BLK_I
VP = N_BLK * SUB


def _pack16(x):
    xi = lax.bitcast_convert_type(x, jnp.uint32)
    lsb = (xi >> 16) & jnp.uint32(1)
    xr = (xi + jnp.uint32(0x7FFF) + lsb) >> 16
    packed = (xr[:HALF] << 16) | xr[HALF:]
    return lax.bitcast_convert_type(packed, jnp.float32)


def _repack_body(xg_ref, xm_ref, out_ref):
    pg = _pack16(xg_ref[...])
    pm = _pack16(xm_ref[...])
    for k in range(SUB // 128):
        xv = jnp.concatenate(
            [p[:, c * SUB + k * 128:c * SUB + k * 128 + 128]
             for c in range(GRP) for p in (pg, pm)], axis=0)
        out_ref[k * 128:(k + 1) * 128, :] = jnp.transpose(xv)


def _repack(tg_t, tm_t):
    return pl.pallas_call(
        _repack_body,
        grid=(N_BLK,),
        in_specs=[pl.BlockSpec((D, BLK_I), lambda i: (0, i)),
                  pl.BlockSpec((D, BLK_I), lambda i: (0, i))],
        out_specs=pl.BlockSpec((SUB, 128), lambda i: (i, 0)),
        out_shape=jax.ShapeDtypeStruct((VP, 128), jnp.float32),
    )(tg_t, tm_t)


def _sc_gather_body(qids_hbm, tbl_hbm, out_hbm, idx_v, rows_v, sem):
    wid = lax.axis_index("s") * NC + lax.axis_index("c")
    pltpu.sync_copy(qids_hbm.at[wid], idx_v)
    copies = []
    for j in range(N_CHUNKS):
        rows = pl.ds(j * IDX_CHUNK, IDX_CHUNK)
        copies.append(pltpu.async_copy(tbl_hbm.at[idx_v.at[j]],
                                       rows_v.at[rows], sem))
    for c in copies:
        c.wait()
    pltpu.sync_copy(rows_v, out_hbm.at[pl.ds(wid * B_PER_W, B_PER_W)])


def _sc_gather(qids, tbl):
    mesh = plsc.VectorSubcoreMesh(core_axis_name="c", subcore_axis_name="s")
    run = pl.kernel(
        _sc_gather_body,
        out_type=jax.ShapeDtypeStruct((B, 128), jnp.float32),
        mesh=mesh,
        scratch_types=[
            pltpu.VMEM((N_CHUNKS, IDX_CHUNK), jnp.int32),
            pltpu.VMEM((B_PER_W, 128), jnp.float32),
            pltpu.SemaphoreType.DMA,
        ],
    )
    return run(qids, tbl)


def _unpack_sel(x128, sel):
    xi = lax.bitcast_convert_type(x128, jnp.uint32)
    lane_w = lax.broadcasted_iota(jnp.int32, xi.shape, 1) // D
    z = jnp.where(lane_w == sel, xi, jnp.uint32(0))
    z = z[:, :64] | z[:, 64:]
    w = z[:, :D] | z[:, D:]
    hi = lax.bitcast_convert_type(w & jnp.uint32(0xFFFF0000), jnp.float32)
    lo = lax.bitcast_convert_type(w << 16, jnp.float32)
    g = jnp.concatenate([hi[:, :HALF], lo[:, :HALF]], axis=1)
    m = jnp.concatenate([hi[:, HALF:], lo[:, HALF:]], axis=1)
    return g, m


def _mlp_body(usel_ref, isel_ref, gu_ref, gi_ref,
              W1u_ref, W1i_ref, b1_ref, W2_ref, b2_ref, W3_ref, b3_ref,
              Wfg_ref, Wfh_ref, bf_ref, out_ref):
    f32 = jnp.float32
    ug, um = _unpack_sel(gu_ref[...], usel_ref[...])
    ig, im = _unpack_sel(gi_ref[...], isel_ref[...])
    gmf = ug * ig
    h = jnp.dot(um, W1u_ref[...], preferred_element_type=f32)
    h += jnp.dot(im, W1i_ref[...], preferred_element_type=f32)
    h = jnp.maximum(h + b1_ref[...], 0.0)
    h = jnp.maximum(jnp.dot(h, W2_ref[...], preferred_element_type=f32)
                    + b2_ref[...], 0.0)
    h = jnp.maximum(jnp.dot(h, W3_ref[...], preferred_element_type=f32)
                    + b3_ref[...], 0.0)
    out = jnp.dot(gmf, Wfg_ref[...], preferred_element_type=f32)
    out += jnp.dot(h, Wfh_ref[...], preferred_element_type=f32)
    out_ref[...] = out + bf_ref[...]


def _mlp(usel, isel, gu, gi, W1, b1, W2, b2, W3, b3, Wf, bf):
    n_blk = 4
    blk = B // n_blk
    full = lambda shape: pl.BlockSpec(shape, lambda i: (0, 0))
    row = lambda w: pl.BlockSpec((blk, w), lambda i: (i, 0))
    grid_spec = pl.GridSpec(
        grid=(n_blk,),
        in_specs=[
            row(1), row(1),
            row(128), row(128),
            full((D, 64)), full((D, 64)), full((1, 64)),
            full((64, 32)), full((1, 32)),
            full((32, 16)), full((1, 16)),
            full((D, 1)), full((16, 1)), full((1, 1)),
        ],
        out_specs=pl.BlockSpec((blk, 1), lambda i: (i, 0)),
    )
    return pl.pallas_call(
        _mlp_body,
        grid_spec=grid_spec,
        out_shape=jax.ShapeDtypeStruct((B, 1), jnp.float32),
    )(usel, isel, gu, gi,
      W1[:D], W1[D:], b1.reshape(1, -1),
      W2, b2.reshape(1, -1),
      W3, b3.reshape(1, -1),
      Wf[:D], Wf[D:], bf.reshape(1, 1))


def kernel(user_ids, item_ids, user_gmf, item_gmf, user_mlp, item_mlp,
           W1, b1, W2, b2, W3, b3, Wf, bf):
    uids = user_ids.astype(jnp.int32)
    iids = item_ids.astype(jnp.int32)
    uq = ((uids // BLK_I) * SUB + uids % SUB).reshape(NW, N_CHUNKS, IDX_CHUNK)
    iq = ((iids // BLK_I) * SUB + iids % SUB).reshape(NW, N_CHUNKS, IDX_CHUNK)
    usel = ((uids // SUB) % GRP).reshape(B, 1)
    isel = ((iids // SUB) % GRP).reshape(B, 1)
    gu = _sc_gather(uq, _repack(user_gmf.T, user_mlp.T))
    gi = _sc_gather(iq, _repack(item_gmf.T, item_mlp.T))
    return _mlp(usel, isel, gu, gi, W1, b1, W2, b2, W3, b3, Wf, bf)
